# baseline (speedup 1.0000x reference)
import functools

import jax
import jax.numpy as jnp
from jax import lax
from jax.experimental import pallas as pl
from jax.experimental.pallas import tpu as pltpu
from jax.experimental.pallas import tpu_sc as plsc

N_PAD = 10240
NC, NS = 2, 16
EDGE_K = 128
D = 128
DEG_W = 128
NBUF = 2
DEG_NBUF = 4
RPT = N_PAD // NS


def _mesh():
    return plsc.VectorSubcoreMesh(
        core_axis_name="c", subcore_axis_name="s", num_cores=NC, num_subcores=NS
    )


def _fill16(ref, rows, width, val):
    v16 = jnp.full((16,), val, jnp.float32)

    def body(t, _):
        ref[t // (width // 16), pl.ds((t % (width // 16)) * 16, 16)] = v16
        return 0

    lax.fori_loop(0, rows * (width // 16), body, 0)


def _zero_acc(acc, zero_v, s):
    base = s * RPT
    for j in range(RPT // EDGE_K):
        pltpu.sync_copy(zero_v, acc.at[pl.ds(base + j * EDGE_K, EDGE_K)])


def _deg_kernel(dst, nchunks):
    @functools.partial(
        pl.kernel,
        out_type=jax.ShapeDtypeStruct((NC, N_PAD, DEG_W), jnp.float32),
        mesh=_mesh(),
        scratch_types=[
            pltpu.VMEM((nchunks, EDGE_K), jnp.int32),
            pltpu.VMEM((EDGE_K, DEG_W), jnp.float32),
            pltpu.VMEM_SHARED((N_PAD, DEG_W), jnp.float32),
        ]
        + [pltpu.SemaphoreType.DMA] * DEG_NBUF,
    )
    def k(dst_hbm, out_hbm, dst_v, ones_v, acc, *sems):
        c = lax.axis_index("c")
        s = lax.axis_index("s")

        pltpu.sync_copy(dst_hbm.at[c, s], dst_v)
        _fill16(ones_v, EDGE_K, DEG_W, 0.0)
        _zero_acc(acc, ones_v, s)
        _fill16(ones_v, EDGE_K, DEG_W, 1.0)
        plsc.subcore_barrier()

        ngroups = (nchunks + DEG_NBUF - 1) // DEG_NBUF

        def group(jj, _):
            t0 = jj * DEG_NBUF
            for b in range(DEG_NBUF):
                t = t0 + b

                @pl.when(t < nchunks)
                def _():
                    @pl.when(t >= DEG_NBUF)
                    def _():
                        pltpu.make_async_copy(
                            ones_v, acc.at[dst_v.at[t - DEG_NBUF]], sems[b]
                        ).wait()

                    pltpu.async_copy(ones_v, acc.at[dst_v.at[t]], sems[b], add=True)

            return 0

        lax.fori_loop(0, ngroups, group, 0)

        for t in range(max(0, nchunks - DEG_NBUF), nchunks):
            pltpu.make_async_copy(ones_v, acc.at[dst_v.at[t]], sems[t % DEG_NBUF]).wait()
        plsc.subcore_barrier()
        pltpu.sync_copy(
            acc.at[pl.ds(s * RPT, RPT)], out_hbm.at[c, pl.ds(s * RPT, RPT)]
        )

    return k(dst)


def _agg_kernel(src, dst, h, nchunks):
    @functools.partial(
        pl.kernel,
        out_type=jax.ShapeDtypeStruct((NC, N_PAD, D), jnp.float32),
        mesh=_mesh(),
        scratch_types=[
            pltpu.VMEM((nchunks, EDGE_K), jnp.int32),
            pltpu.VMEM((nchunks, EDGE_K), jnp.int32),
            pltpu.VMEM((NBUF, EDGE_K, D), jnp.float32),
            pltpu.VMEM_SHARED((N_PAD, D), jnp.float32),
        ]
        + [pltpu.SemaphoreType.DMA] * (2 * NBUF),
    )
    def k(src_hbm, dst_hbm, h_hbm, out_hbm, src_v, dst_v, rows_v, acc, *sems):
        gsem = sems[:NBUF]
        ssem = sems[NBUF:]
        c = lax.axis_index("c")
        s = lax.axis_index("s")

        pltpu.sync_copy(src_hbm.at[c, s], src_v)
        pltpu.sync_copy(dst_hbm.at[c, s], dst_v)
        _fill16(rows_v.at[0], EDGE_K, D, 0.0)
        _zero_acc(acc, rows_v.at[0], s)
        plsc.subcore_barrier()

        def gather(t, b):
            pltpu.async_copy(h_hbm.at[src_v.at[t]], rows_v.at[b], gsem[b])

        def scatter(t, b):
            pltpu.async_copy(rows_v.at[b], acc.at[dst_v.at[t]], ssem[b], add=True)

        gather(0, 0)

        ngroups = (nchunks + NBUF - 1) // NBUF

        def group(jj, _):
            t0 = jj * NBUF
            for b in range(NBUF):
                t = t0 + b

                @pl.when(t < nchunks)
                def _():
                    pltpu.make_async_copy(
                        h_hbm.at[src_v.at[t]], rows_v.at[b], gsem[b]
                    ).wait()
                    b1 = 1 - b

                    @pl.when(t >= 1)
                    def _():
                        pltpu.make_async_copy(
                            rows_v.at[b1], acc.at[dst_v.at[t - 1]], ssem[b1]
                        ).wait()

                    @pl.when(t + 1 < nchunks)
                    def _():
                        gather(t + 1, b1)

                    scatter(t, b)

            return 0

        lax.fori_loop(0, ngroups, group, 0)

        bl = (nchunks - 1) % NBUF
        pltpu.make_async_copy(
            rows_v.at[bl], acc.at[dst_v.at[nchunks - 1]], ssem[bl]
        ).wait()
        plsc.subcore_barrier()
        pltpu.sync_copy(
            acc.at[pl.ds(s * RPT, RPT)], out_hbm.at[c, pl.ds(s * RPT, RPT)]
        )

    return k(src, dst, h)


_BM = 1024


def _mm_scale(x, w, deg0, deg1):
    m, kdim = x.shape
    _, d = w.shape

    def body(x_ref, w_ref, d0_ref, d1_ref, o_ref):
        dis = lax.rsqrt(d0_ref[...] + d1_ref[...] + 1.0)
        o_ref[...] = (
            jnp.dot(x_ref[...], w_ref[...], preferred_element_type=jnp.float32) * dis
        )

    return pl.pallas_call(
        body,
        grid=(m // _BM,),
        in_specs=[
            pl.BlockSpec((_BM, kdim), lambda i: (i, 0)),
            pl.BlockSpec((kdim, d), lambda i: (0, 0)),
            pl.BlockSpec((_BM, 1), lambda i: (i, 0)),
            pl.BlockSpec((_BM, 1), lambda i: (i, 0)),
        ],
        out_specs=pl.BlockSpec((_BM, d), lambda i: (i, 0)),
        out_shape=jax.ShapeDtypeStruct((m, d), jnp.float32),
    )(x, w, deg0, deg1)


def _combine_relu_mm_scale(p, h1, deg0, deg1, b1, w2):
    m, d = h1.shape
    _, d2 = w2.shape

    def body(p_ref, h1_ref, d0_ref, d1_ref, b_ref, w_ref, o_ref):
        dis = lax.rsqrt(d0_ref[...] + d1_ref[...] + 1.0)
        z = jnp.maximum(
            (p_ref[0] + p_ref[1] + h1_ref[...]) * dis + b_ref[...], 0.0
        )
        r = jnp.dot(z, w_ref[...], preferred_element_type=jnp.float32) * dis
        o_ref[...] = jnp.concatenate(
            [r, jnp.zeros((_BM, d - d2), jnp.float32)], axis=1
        )

    return pl.pallas_call(
        body,
        grid=(m // _BM,),
        in_specs=[
            pl.BlockSpec((NC, _BM, d), lambda i: (0, i, 0)),
            pl.BlockSpec((_BM, d), lambda i: (i, 0)),
            pl.BlockSpec((_BM, 1), lambda i: (i, 0)),
            pl.BlockSpec((_BM, 1), lambda i: (i, 0)),
            pl.BlockSpec((1, d), lambda i: (0, 0)),
            pl.BlockSpec((d, d2), lambda i: (0, 0)),
        ],
        out_specs=pl.BlockSpec((_BM, d), lambda i: (i, 0)),
        out_shape=jax.ShapeDtypeStruct((m, d), jnp.float32),
    )(p, h1, deg0, deg1, b1, w2)


def _combine_bias(p, h2, deg0, deg1, b2, d2):
    m, d = h2.shape

    def body(p_ref, h2_ref, d0_ref, d1_ref, b_ref, o_ref):
        dis = lax.rsqrt(d0_ref[...] + d1_ref[...] + 1.0)
        o_ref[...] = (
            p_ref[0][:, :d2] + p_ref[1][:, :d2] + h2_ref[:, :d2]
        ) * dis + b_ref[...]

    return pl.pallas_call(
        body,
        grid=(m // _BM,),
        in_specs=[
            pl.BlockSpec((NC, _BM, d), lambda i: (0, i, 0)),
            pl.BlockSpec((_BM, d), lambda i: (i, 0)),
            pl.BlockSpec((_BM, 1), lambda i: (i, 0)),
            pl.BlockSpec((_BM, 1), lambda i: (i, 0)),
            pl.BlockSpec((1, d2), lambda i: (0, 0)),
        ],
        out_specs=pl.BlockSpec((_BM, d2), lambda i: (i, 0)),
        out_shape=jax.ShapeDtypeStruct((m, d2), jnp.float32),
    )(p, h2, deg0, deg1, b2)


def kernel(x, edge_index, W1, b1, W2, b2):
    n, f_in = x.shape
    e = edge_index.shape[1]

    epw = NC * NS * EDGE_K
    e_pad = ((e + epw - 1) // epw) * epw
    nchunks = e_pad // (NC * NS * EDGE_K)
    src = jnp.full((e_pad,), n, jnp.int32).at[:e].set(
        edge_index[0].astype(jnp.int32)
    )
    dst = (n + (jnp.arange(e_pad, dtype=jnp.int32) % (N_PAD - n))).at[:e].set(
        edge_index[1].astype(jnp.int32)
    )
    src = src.reshape(NC, NS, nchunks, EDGE_K)
    dst = dst.reshape(NC, NS, nchunks, EDGE_K)

    xp = jnp.zeros((N_PAD, f_in), x.dtype).at[:n].set(x)

    degp = _deg_kernel(dst, nchunks)
    deg0 = degp[0, :, 0:1]
    deg1 = degp[1, :, 0:1]

    h1 = _mm_scale(xp, W1, deg0, deg1)
    agg1 = _agg_kernel(src, dst, h1, nchunks)
    h2 = _combine_relu_mm_scale(agg1, h1, deg0, deg1, b1.reshape(1, -1), W2)
    agg2 = _agg_kernel(src, dst, h2, nchunks)
    out = _combine_bias(agg2, h2, deg0, deg1, b2.reshape(1, -1), W2.shape[1])
    return out[:n]


# speedup vs baseline: 1.0945x; 1.0945x over previous
import functools

import jax
import jax.numpy as jnp
from jax import lax
from jax.experimental import pallas as pl
from jax.experimental.pallas import tpu as pltpu
from jax.experimental.pallas import tpu_sc as plsc

N_PAD = 10240
NC, NS = 2, 16
EDGE_K = 128
D = 128
DEG_W = 128
NBUF = 4
DEG_NBUF = 4
RPT = N_PAD // NS


def _mesh():
    return plsc.VectorSubcoreMesh(
        core_axis_name="c", subcore_axis_name="s", num_cores=NC, num_subcores=NS
    )


def _fill16(ref, rows, width, val):
    v16 = jnp.full((16,), val, jnp.float32)

    def body(t, _):
        ref[t // (width // 16), pl.ds((t % (width // 16)) * 16, 16)] = v16
        return 0

    lax.fori_loop(0, rows * (width // 16), body, 0)


def _zero_acc(acc, zero_v, s, rows=None):
    rows = EDGE_K if rows is None else rows
    base = s * RPT
    for j in range(RPT // rows):
        pltpu.sync_copy(zero_v, acc.at[pl.ds(base + j * rows, rows)])


def _deg_kernel(dst, nchunks):
    @functools.partial(
        pl.kernel,
        out_type=jax.ShapeDtypeStruct((NC, N_PAD, DEG_W), jnp.float32),
        mesh=_mesh(),
        scratch_types=[
            pltpu.VMEM((nchunks, EDGE_K), jnp.int32),
            pltpu.VMEM((EDGE_K, DEG_W), jnp.float32),
            pltpu.VMEM_SHARED((N_PAD, DEG_W), jnp.float32),
        ]
        + [pltpu.SemaphoreType.DMA] * DEG_NBUF,
    )
    def k(dst_hbm, out_hbm, dst_v, ones_v, acc, *sems):
        c = lax.axis_index("c")
        s = lax.axis_index("s")

        pltpu.sync_copy(dst_hbm.at[c, s], dst_v)
        _fill16(ones_v, EDGE_K, DEG_W, 0.0)
        _zero_acc(acc, ones_v, s)
        _fill16(ones_v, EDGE_K, DEG_W, 1.0)
        plsc.subcore_barrier()

        ngroups = (nchunks + DEG_NBUF - 1) // DEG_NBUF

        def group(jj, _):
            t0 = jj * DEG_NBUF
            for b in range(DEG_NBUF):
                t = t0 + b

                @pl.when(t < nchunks)
                def _():
                    @pl.when(t >= DEG_NBUF)
                    def _():
                        pltpu.make_async_copy(
                            ones_v, acc.at[dst_v.at[t - DEG_NBUF]], sems[b]
                        ).wait()

                    pltpu.async_copy(ones_v, acc.at[dst_v.at[t]], sems[b], add=True)

            return 0

        lax.fori_loop(0, ngroups, group, 0)

        for t in range(max(0, nchunks - DEG_NBUF), nchunks):
            pltpu.make_async_copy(ones_v, acc.at[dst_v.at[t]], sems[t % DEG_NBUF]).wait()
        plsc.subcore_barrier()
        pltpu.sync_copy(
            acc.at[pl.ds(s * RPT, RPT)], out_hbm.at[c, pl.ds(s * RPT, RPT)]
        )

    return k(dst)


AGG_K = 64
QD = 8


def _agg_kernel(pk, h, nchunks):
    @functools.partial(
        pl.kernel,
        out_type=jax.ShapeDtypeStruct((NC, N_PAD, D), jnp.float32),
        mesh=_mesh(),
        scratch_types=[
            pltpu.VMEM((nchunks // 2, 128), jnp.int32),
            pltpu.VMEM((QD, AGG_K), jnp.int32),
            pltpu.VMEM((QD, AGG_K), jnp.int32),
            pltpu.VMEM((NBUF, AGG_K, D), jnp.float32),
            pltpu.VMEM_SHARED((N_PAD, D), jnp.float32),
        ]
        + [pltpu.SemaphoreType.DMA] * (2 * NBUF),
    )
    def k(pk_hbm, h_hbm, out_hbm, pk_v, src_u, dst_u, rows_v, acc, *sems):
        gsem = sems[:NBUF]
        ssem = sems[NBUF:]
        c = lax.axis_index("c")
        s = lax.axis_index("s")

        pltpu.sync_copy(pk_hbm.at[c, s], pk_v)
        _fill16(rows_v.at[0], AGG_K, D, 0.0)
        _zero_acc(acc, rows_v.at[0], s, AGG_K)
        plsc.subcore_barrier()

        def unpack(t):
            q = t % QD
            r = t // 2
            o = (t % 2) * AGG_K

            for l in range(AGG_K // 16):
                v = pk_v[r, pl.ds(o + l * 16, 16)]
                src_u[q, pl.ds(l * 16, 16)] = jnp.bitwise_and(v, 0xFFFF)
                dst_u[q, pl.ds(l * 16, 16)] = jnp.right_shift(v, 16)

        def gather(t, b):
            pltpu.async_copy(h_hbm.at[src_u.at[t % QD]], rows_v.at[b], gsem[b])

        def scatter(t, b):
            pltpu.async_copy(
                rows_v.at[b], acc.at[dst_u.at[t % QD]], ssem[b], add=True
            )

        for t in range(min(3, nchunks)):
            unpack(t)
            gather(t, t)

        ngroups = (nchunks + NBUF - 1) // NBUF

        def group(jj, _):
            t0 = jj * NBUF
            for b in range(NBUF):
                t = t0 + b

                @pl.when(t < nchunks)
                def _():
                    pltpu.make_async_copy(
                        h_hbm.at[src_u.at[t % QD]], rows_v.at[b], gsem[b]
                    ).wait()
                    b3 = (b + 3) % NBUF

                    @pl.when(t >= 1)
                    def _():
                        pltpu.make_async_copy(
                            rows_v.at[b3], acc.at[dst_u.at[(t - 1) % QD]], ssem[b3]
                        ).wait()

                    @pl.when(t + 3 < nchunks)
                    def _():
                        unpack(t + 3)
                        gather(t + 3, b3)

                    scatter(t, b)

            return 0

        lax.fori_loop(0, ngroups, group, 0)

        bl = (nchunks - 1) % NBUF
        pltpu.make_async_copy(
            rows_v.at[bl], acc.at[dst_u.at[(nchunks - 1) % QD]], ssem[bl]
        ).wait()
        plsc.subcore_barrier()
        pltpu.sync_copy(
            acc.at[pl.ds(s * RPT, RPT)], out_hbm.at[c, pl.ds(s * RPT, RPT)]
        )

    return k(pk, h)


_BM = 1024


def _mm_scale(x, w, deg0, deg1):
    m, kdim = x.shape
    _, d = w.shape

    def body(x_ref, w_ref, d0_ref, d1_ref, o_ref):
        dis = lax.rsqrt(d0_ref[...] + d1_ref[...] + 1.0)
        o_ref[...] = (
            jnp.dot(x_ref[...], w_ref[...], preferred_element_type=jnp.float32) * dis
        )

    return pl.pallas_call(
        body,
        grid=(m // _BM,),
        in_specs=[
            pl.BlockSpec((_BM, kdim), lambda i: (i, 0)),
            pl.BlockSpec((kdim, d), lambda i: (0, 0)),
            pl.BlockSpec((_BM, 1), lambda i: (i, 0)),
            pl.BlockSpec((_BM, 1), lambda i: (i, 0)),
        ],
        out_specs=pl.BlockSpec((_BM, d), lambda i: (i, 0)),
        out_shape=jax.ShapeDtypeStruct((m, d), jnp.float32),
    )(x, w, deg0, deg1)


def _combine_relu_mm_scale(p, h1, deg0, deg1, b1, w2):
    m, d = h1.shape
    _, d2 = w2.shape

    def body(p_ref, h1_ref, d0_ref, d1_ref, b_ref, w_ref, o_ref):
        dis = lax.rsqrt(d0_ref[...] + d1_ref[...] + 1.0)
        z = jnp.maximum(
            (p_ref[0] + p_ref[1] + h1_ref[...]) * dis + b_ref[...], 0.0
        )
        r = jnp.dot(z, w_ref[...], preferred_element_type=jnp.float32) * dis
        o_ref[...] = jnp.concatenate(
            [r, jnp.zeros((_BM, d - d2), jnp.float32)], axis=1
        )

    return pl.pallas_call(
        body,
        grid=(m // _BM,),
        in_specs=[
            pl.BlockSpec((NC, _BM, d), lambda i: (0, i, 0)),
            pl.BlockSpec((_BM, d), lambda i: (i, 0)),
            pl.BlockSpec((_BM, 1), lambda i: (i, 0)),
            pl.BlockSpec((_BM, 1), lambda i: (i, 0)),
            pl.BlockSpec((1, d), lambda i: (0, 0)),
            pl.BlockSpec((d, d2), lambda i: (0, 0)),
        ],
        out_specs=pl.BlockSpec((_BM, d), lambda i: (i, 0)),
        out_shape=jax.ShapeDtypeStruct((m, d), jnp.float32),
    )(p, h1, deg0, deg1, b1, w2)


def _combine_bias(p, h2, deg0, deg1, b2, d2):
    m, d = h2.shape

    def body(p_ref, h2_ref, d0_ref, d1_ref, b_ref, o_ref):
        dis = lax.rsqrt(d0_ref[...] + d1_ref[...] + 1.0)
        o_ref[...] = (
            p_ref[0][:, :d2] + p_ref[1][:, :d2] + h2_ref[:, :d2]
        ) * dis + b_ref[...]

    return pl.pallas_call(
        body,
        grid=(m // _BM,),
        in_specs=[
            pl.BlockSpec((NC, _BM, d), lambda i: (0, i, 0)),
            pl.BlockSpec((_BM, d), lambda i: (i, 0)),
            pl.BlockSpec((_BM, 1), lambda i: (i, 0)),
            pl.BlockSpec((_BM, 1), lambda i: (i, 0)),
            pl.BlockSpec((1, d2), lambda i: (0, 0)),
        ],
        out_specs=pl.BlockSpec((_BM, d2), lambda i: (i, 0)),
        out_shape=jax.ShapeDtypeStruct((m, d2), jnp.float32),
    )(p, h2, deg0, deg1, b2)


def kernel(x, edge_index, W1, b1, W2, b2):
    n, f_in = x.shape
    e = edge_index.shape[1]

    epw = NC * NS * EDGE_K
    e_pad = ((e + epw - 1) // epw) * epw
    nchunks = e_pad // (NC * NS * EDGE_K)
    nchunks_a = e_pad // (NC * NS * AGG_K)
    src = jnp.full((e_pad,), n, jnp.int32).at[:e].set(
        edge_index[0].astype(jnp.int32)
    )
    dst = (n + (jnp.arange(e_pad, dtype=jnp.int32) % (N_PAD - n))).at[:e].set(
        edge_index[1].astype(jnp.int32)
    )
    pk = jnp.bitwise_or(
        src, jnp.left_shift(dst, 16)
    ).reshape(NC, NS, nchunks, EDGE_K)
    dst = dst.reshape(NC, NS, nchunks, EDGE_K)

    xp = jnp.zeros((N_PAD, f_in), x.dtype).at[:n].set(x)

    degp = _deg_kernel(dst, nchunks)
    deg0 = degp[0, :, 0:1]
    deg1 = degp[1, :, 0:1]

    h1 = _mm_scale(xp, W1, deg0, deg1)
    agg1 = _agg_kernel(pk, h1, nchunks_a)
    h2 = _combine_relu_mm_scale(agg1, h1, deg0, deg1, b1.reshape(1, -1), W2)
    agg2 = _agg_kernel(pk, h2, nchunks_a)
    out = _combine_bias(agg2, h2, deg0, deg1, b2.reshape(1, -1), W2.shape[1])
    return out[:n]


# speedup vs baseline: 1.0946x; 1.0001x over previous
import functools

import jax
import jax.numpy as jnp
from jax import lax
from jax.experimental import pallas as pl
from jax.experimental.pallas import tpu as pltpu
from jax.experimental.pallas import tpu_sc as plsc

N_PAD = 10240
NC, NS = 2, 16
EDGE_K = 128
D = 128
DEG_W = 128
NBUF = 4
DEG_NBUF = 4
RPT = N_PAD // NS


def _mesh():
    return plsc.VectorSubcoreMesh(
        core_axis_name="c", subcore_axis_name="s", num_cores=NC, num_subcores=NS
    )


def _fill16(ref, rows, width, val):
    v16 = jnp.full((16,), val, jnp.float32)

    def body(t, _):
        ref[t // (width // 16), pl.ds((t % (width // 16)) * 16, 16)] = v16
        return 0

    lax.fori_loop(0, rows * (width // 16), body, 0)


def _zero_acc(acc, zero_v, s, rows=None):
    rows = EDGE_K if rows is None else rows
    base = s * RPT
    for j in range(RPT // rows):
        pltpu.sync_copy(zero_v, acc.at[pl.ds(base + j * rows, rows)])


def _deg_kernel(dst, nchunks):
    @functools.partial(
        pl.kernel,
        out_type=jax.ShapeDtypeStruct((NC, N_PAD, DEG_W), jnp.float32),
        mesh=_mesh(),
        scratch_types=[
            pltpu.VMEM((nchunks, EDGE_K), jnp.int32),
            pltpu.VMEM((EDGE_K, DEG_W), jnp.float32),
            pltpu.VMEM_SHARED((N_PAD, DEG_W), jnp.float32),
        ]
        + [pltpu.SemaphoreType.DMA] * DEG_NBUF,
    )
    def k(dst_hbm, out_hbm, dst_v, ones_v, acc, *sems):
        c = lax.axis_index("c")
        s = lax.axis_index("s")

        pltpu.sync_copy(dst_hbm.at[c, s], dst_v)
        _fill16(ones_v, EDGE_K, DEG_W, 0.0)
        _zero_acc(acc, ones_v, s)
        _fill16(ones_v, EDGE_K, DEG_W, 1.0)
        plsc.subcore_barrier()

        ngroups = (nchunks + DEG_NBUF - 1) // DEG_NBUF

        def group(jj, _):
            t0 = jj * DEG_NBUF
            for b in range(DEG_NBUF):
                t = t0 + b

                @pl.when(t < nchunks)
                def _():
                    @pl.when(t >= DEG_NBUF)
                    def _():
                        pltpu.make_async_copy(
                            ones_v, acc.at[dst_v.at[t - DEG_NBUF]], sems[b]
                        ).wait()

                    pltpu.async_copy(ones_v, acc.at[dst_v.at[t]], sems[b], add=True)

            return 0

        lax.fori_loop(0, ngroups, group, 0)

        for t in range(max(0, nchunks - DEG_NBUF), nchunks):
            pltpu.make_async_copy(ones_v, acc.at[dst_v.at[t]], sems[t % DEG_NBUF]).wait()
        plsc.subcore_barrier()
        pltpu.sync_copy(
            acc.at[pl.ds(s * RPT, RPT)], out_hbm.at[c, pl.ds(s * RPT, RPT)]
        )

    return k(dst)


AGG_K = 64
QD = 8


def _agg_kernel(pk, h, nchunks):
    @functools.partial(
        pl.kernel,
        out_type=jax.ShapeDtypeStruct((NC, N_PAD, D), jnp.float32),
        mesh=_mesh(),
        scratch_types=[
            pltpu.VMEM((nchunks // 2, 128), jnp.int32),
            pltpu.VMEM((QD, AGG_K), jnp.int32),
            pltpu.VMEM((QD, AGG_K), jnp.int32),
            pltpu.VMEM((NBUF, AGG_K, D), jnp.float32),
            pltpu.VMEM_SHARED((N_PAD, D), jnp.float32),
        ]
        + [pltpu.SemaphoreType.DMA] * (2 * NBUF),
    )
    def k(pk_hbm, h_hbm, out_hbm, pk_v, src_u, dst_u, rows_v, acc, *sems):
        gsem = sems[:NBUF]
        ssem = sems[NBUF:]
        c = lax.axis_index("c")
        s = lax.axis_index("s")

        pltpu.sync_copy(pk_hbm.at[c, s], pk_v)
        _fill16(rows_v.at[0], AGG_K, D, 0.0)
        _zero_acc(acc, rows_v.at[0], s, AGG_K)
        plsc.subcore_barrier()

        def unpack(t):
            q = t % QD
            r = t // 2
            o = (t % 2) * AGG_K

            for l in range(AGG_K // 16):
                v = pk_v[r, pl.ds(o + l * 16, 16)]
                src_u[q, pl.ds(l * 16, 16)] = jnp.bitwise_and(v, 0xFFFF)
                dst_u[q, pl.ds(l * 16, 16)] = jnp.right_shift(v, 16)

        def gather(t, b):
            pltpu.async_copy(h_hbm.at[src_u.at[t % QD]], rows_v.at[b], gsem[b])

        def scatter(t, b):
            pltpu.async_copy(
                rows_v.at[b], acc.at[dst_u.at[t % QD]], ssem[b], add=True
            )

        LA = NBUF - 1
        for t in range(min(LA, nchunks)):
            unpack(t)
            gather(t, t)

        ngroups = (nchunks + NBUF - 1) // NBUF

        def group(jj, _):
            t0 = jj * NBUF
            for b in range(NBUF):
                t = t0 + b

                @pl.when(t < nchunks)
                def _():
                    pltpu.make_async_copy(
                        h_hbm.at[src_u.at[t % QD]], rows_v.at[b], gsem[b]
                    ).wait()
                    bl_ = (b + LA) % NBUF

                    @pl.when(t >= 1)
                    def _():
                        pltpu.make_async_copy(
                            rows_v.at[bl_], acc.at[dst_u.at[(t - 1) % QD]], ssem[bl_]
                        ).wait()

                    @pl.when(t + LA < nchunks)
                    def _():
                        unpack(t + LA)
                        gather(t + LA, bl_)

                    scatter(t, b)

            return 0

        lax.fori_loop(0, ngroups, group, 0)

        bl = (nchunks - 1) % NBUF
        pltpu.make_async_copy(
            rows_v.at[bl], acc.at[dst_u.at[(nchunks - 1) % QD]], ssem[bl]
        ).wait()
        plsc.subcore_barrier()
        pltpu.sync_copy(
            acc.at[pl.ds(s * RPT, RPT)], out_hbm.at[c, pl.ds(s * RPT, RPT)]
        )

    return k(pk, h)


_BM = 1024


def _mm_scale(x, w, deg0, deg1):
    m, kdim = x.shape
    _, d = w.shape

    def body(x_ref, w_ref, d0_ref, d1_ref, o_ref):
        dis = lax.rsqrt(d0_ref[...] + d1_ref[...] + 1.0)
        o_ref[...] = (
            jnp.dot(x_ref[...], w_ref[...], preferred_element_type=jnp.float32) * dis
        )

    return pl.pallas_call(
        body,
        grid=(m // _BM,),
        in_specs=[
            pl.BlockSpec((_BM, kdim), lambda i: (i, 0)),
            pl.BlockSpec((kdim, d), lambda i: (0, 0)),
            pl.BlockSpec((_BM, 1), lambda i: (i, 0)),
            pl.BlockSpec((_BM, 1), lambda i: (i, 0)),
        ],
        out_specs=pl.BlockSpec((_BM, d), lambda i: (i, 0)),
        out_shape=jax.ShapeDtypeStruct((m, d), jnp.float32),
    )(x, w, deg0, deg1)


def _combine_relu_mm_scale(p, h1, deg0, deg1, b1, w2):
    m, d = h1.shape
    _, d2 = w2.shape

    def body(p_ref, h1_ref, d0_ref, d1_ref, b_ref, w_ref, o_ref):
        dis = lax.rsqrt(d0_ref[...] + d1_ref[...] + 1.0)
        z = jnp.maximum(
            (p_ref[0] + p_ref[1] + h1_ref[...]) * dis + b_ref[...], 0.0
        )
        r = jnp.dot(z, w_ref[...], preferred_element_type=jnp.float32) * dis
        o_ref[...] = jnp.concatenate(
            [r, jnp.zeros((_BM, d - d2), jnp.float32)], axis=1
        )

    return pl.pallas_call(
        body,
        grid=(m // _BM,),
        in_specs=[
            pl.BlockSpec((NC, _BM, d), lambda i: (0, i, 0)),
            pl.BlockSpec((_BM, d), lambda i: (i, 0)),
            pl.BlockSpec((_BM, 1), lambda i: (i, 0)),
            pl.BlockSpec((_BM, 1), lambda i: (i, 0)),
            pl.BlockSpec((1, d), lambda i: (0, 0)),
            pl.BlockSpec((d, d2), lambda i: (0, 0)),
        ],
        out_specs=pl.BlockSpec((_BM, d), lambda i: (i, 0)),
        out_shape=jax.ShapeDtypeStruct((m, d), jnp.float32),
    )(p, h1, deg0, deg1, b1, w2)


def _combine_bias(p, h2, deg0, deg1, b2, d2):
    m, d = h2.shape

    def body(p_ref, h2_ref, d0_ref, d1_ref, b_ref, o_ref):
        dis = lax.rsqrt(d0_ref[...] + d1_ref[...] + 1.0)
        o_ref[...] = (
            p_ref[0][:, :d2] + p_ref[1][:, :d2] + h2_ref[:, :d2]
        ) * dis + b_ref[...]

    return pl.pallas_call(
        body,
        grid=(m // _BM,),
        in_specs=[
            pl.BlockSpec((NC, _BM, d), lambda i: (0, i, 0)),
            pl.BlockSpec((_BM, d), lambda i: (i, 0)),
            pl.BlockSpec((_BM, 1), lambda i: (i, 0)),
            pl.BlockSpec((_BM, 1), lambda i: (i, 0)),
            pl.BlockSpec((1, d2), lambda i: (0, 0)),
        ],
        out_specs=pl.BlockSpec((_BM, d2), lambda i: (i, 0)),
        out_shape=jax.ShapeDtypeStruct((m, d2), jnp.float32),
    )(p, h2, deg0, deg1, b2)


def kernel(x, edge_index, W1, b1, W2, b2):
    n, f_in = x.shape
    e = edge_index.shape[1]

    epw = NC * NS * EDGE_K
    e_pad = ((e + epw - 1) // epw) * epw
    nchunks = e_pad // (NC * NS * EDGE_K)
    nchunks_a = e_pad // (NC * NS * AGG_K)
    src = jnp.full((e_pad,), n, jnp.int32).at[:e].set(
        edge_index[0].astype(jnp.int32)
    )
    dst = (n + (jnp.arange(e_pad, dtype=jnp.int32) % (N_PAD - n))).at[:e].set(
        edge_index[1].astype(jnp.int32)
    )
    pk = jnp.bitwise_or(
        src, jnp.left_shift(dst, 16)
    ).reshape(NC, NS, nchunks, EDGE_K)
    dst = dst.reshape(NC, NS, nchunks, EDGE_K)

    xp = jnp.zeros((N_PAD, f_in), x.dtype).at[:n].set(x)

    degp = _deg_kernel(dst, nchunks)
    deg0 = degp[0, :, 0:1]
    deg1 = degp[1, :, 0:1]

    h1 = _mm_scale(xp, W1, deg0, deg1)
    agg1 = _agg_kernel(pk, h1, nchunks_a)
    h2 = _combine_relu_mm_scale(agg1, h1, deg0, deg1, b1.reshape(1, -1), W2)
    agg2 = _agg_kernel(pk, h2, nchunks_a)
    out = _combine_bias(agg2, h2, deg0, deg1, b2.reshape(1, -1), W2.shape[1])
    return out[:n]
